# baseline (speedup 1.0000x reference)
import functools

import jax
import jax.numpy as jnp
from jax.experimental import pallas as pl


def _xw_kernel(x_ref, w_ref, o_ref):
    o_ref[...] = jnp.dot(x_ref[...], w_ref[...],
                         preferred_element_type=jnp.float32)


def _layer1_kernel(adj_ref, x1_ref, b1_ref, w2_ref, z_ref):
    y = jnp.dot(adj_ref[...], x1_ref[...], preferred_element_type=jnp.float32)
    h = jnp.maximum(y + b1_ref[...], 0.0)
    z_ref[...] = jnp.dot(h, w2_ref[...], preferred_element_type=jnp.float32)


def _layer2_kernel(adj_ref, z_ref, b2_ref, o_ref):
    o = jnp.dot(adj_ref[...], z_ref[...], preferred_element_type=jnp.float32)
    o = o + b2_ref[...]
    m = jnp.max(o, axis=1, keepdims=True)
    e = o - m
    o_ref[...] = e - jnp.log(jnp.sum(jnp.exp(e), axis=1, keepdims=True))


@functools.partial(jax.jit, static_argnames=())
def kernel(feature, adj, W1, b1, W2, b2):
    n, f_in = feature.shape
    h_dim = W1.shape[1]
    c_dim = W2.shape[1]

    bi = 400
    grid = n // bi

    x1 = pl.pallas_call(
        _xw_kernel,
        grid=(grid,),
        in_specs=[
            pl.BlockSpec((bi, f_in), lambda i: (i, 0)),
            pl.BlockSpec((f_in, h_dim), lambda i: (0, 0)),
        ],
        out_specs=pl.BlockSpec((bi, h_dim), lambda i: (i, 0)),
        out_shape=jax.ShapeDtypeStruct((n, h_dim), jnp.float32),
    )(feature, W1)

    b1_2d = b1.reshape(1, h_dim)
    b2_2d = b2.reshape(1, c_dim)

    z = pl.pallas_call(
        _layer1_kernel,
        grid=(grid,),
        in_specs=[
            pl.BlockSpec((bi, n), lambda i: (i, 0)),
            pl.BlockSpec((n, h_dim), lambda i: (0, 0)),
            pl.BlockSpec((1, h_dim), lambda i: (0, 0)),
            pl.BlockSpec((h_dim, c_dim), lambda i: (0, 0)),
        ],
        out_specs=pl.BlockSpec((bi, c_dim), lambda i: (i, 0)),
        out_shape=jax.ShapeDtypeStruct((n, c_dim), jnp.float32),
    )(adj, x1, b1_2d, W2)

    out = pl.pallas_call(
        _layer2_kernel,
        grid=(grid,),
        in_specs=[
            pl.BlockSpec((bi, n), lambda i: (i, 0)),
            pl.BlockSpec((n, c_dim), lambda i: (0, 0)),
            pl.BlockSpec((1, c_dim), lambda i: (0, 0)),
        ],
        out_specs=pl.BlockSpec((bi, c_dim), lambda i: (i, 0)),
        out_shape=jax.ShapeDtypeStruct((n, c_dim), jnp.float32),
    )(adj, z, b2_2d)

    return out
